# Initial kernel scaffold; baseline (speedup 1.0000x reference)
import jax
import jax.numpy as jnp
from jax.experimental import pallas as pl


def kernel(eeg_nodes, eeg_idx, W1_0, b1_0, W2_0, b2_0, W1_1, b1_1, W2_1, b2_1, W1_2, b1_2, W2_2, b2_2, W1_3, b1_3, W2_3, b2_3, Wm, bm, Wv, bv):
    raise NotImplementedError("write your pallas kernel here")


# speedup vs baseline: 1.0005x; 1.0005x over previous
import functools

import jax
import jax.numpy as jnp
from jax.experimental import pallas as pl
from jax.experimental.pallas import tpu as pltpu


def _pick_mb(n):
    for mb in (400, 256, 200, 128, 80, 40, 16, 8):
        if n % mb == 0:
            return mb
    return n


def _gin_mlp_kernel(x_ref, a_ref, w1_ref, b1_ref, w2_ref, b2_ref, o_ref, *, act):
    h = x_ref[...] + a_ref[...]
    h = jnp.dot(h, w1_ref[...], preferred_element_type=jnp.float32) + b1_ref[...]
    h = jnp.maximum(h, 0.0)
    y = jnp.dot(h, w2_ref[...], preferred_element_type=jnp.float32) + b2_ref[...]
    if act:
        y = jnp.maximum(y, 0.0)
    o_ref[...] = y


def _gin_mlp(x, agg, w1, b1, w2, b2, act):
    n, d_in = x.shape
    h = w1.shape[1]
    d_out = w2.shape[1]
    mb = _pick_mb(n)
    grid = (n // mb,)
    return pl.pallas_call(
        functools.partial(_gin_mlp_kernel, act=act),
        grid=grid,
        in_specs=[
            pl.BlockSpec((mb, d_in), lambda i: (i, 0)),
            pl.BlockSpec((mb, d_in), lambda i: (i, 0)),
            pl.BlockSpec((d_in, h), lambda i: (0, 0)),
            pl.BlockSpec((1, h), lambda i: (0, 0)),
            pl.BlockSpec((h, d_out), lambda i: (0, 0)),
            pl.BlockSpec((1, d_out), lambda i: (0, 0)),
        ],
        out_specs=pl.BlockSpec((mb, d_out), lambda i: (i, 0)),
        out_shape=jax.ShapeDtypeStruct((n, d_out), jnp.float32),
    )(x, agg, w1, b1.reshape(1, -1), w2, b2.reshape(1, -1))


def _final_kernel(x_ref, a_ref, w1_ref, b1_ref, w2_ref, b2_ref,
                  wm_ref, bm_ref, wv_ref, bv_ref, eps_ref,
                  z_ref, m_ref, v_ref):
    h = x_ref[...] + a_ref[...]
    h = jnp.dot(h, w1_ref[...], preferred_element_type=jnp.float32) + b1_ref[...]
    h = jnp.maximum(h, 0.0)
    y = jnp.dot(h, w2_ref[...], preferred_element_type=jnp.float32) + b2_ref[...]
    mean = jnp.dot(y, wm_ref[...], preferred_element_type=jnp.float32) + bm_ref[...]
    var = jnp.dot(y, wv_ref[...], preferred_element_type=jnp.float32) + bv_ref[...]
    m_ref[...] = mean
    v_ref[...] = var
    z_ref[...] = mean + var * eps_ref[...]


def _final_layer(x, agg, w1, b1, w2, b2, wm, bm, wv, bv, eps):
    n, d_in = x.shape
    h = w1.shape[1]
    d_out = w2.shape[1]
    lat = wm.shape[1]
    mb = _pick_mb(n)
    grid = (n // mb,)
    shp = functools.partial(jax.ShapeDtypeStruct, dtype=jnp.float32)
    return pl.pallas_call(
        _final_kernel,
        grid=grid,
        in_specs=[
            pl.BlockSpec((mb, d_in), lambda i: (i, 0)),
            pl.BlockSpec((mb, d_in), lambda i: (i, 0)),
            pl.BlockSpec((d_in, h), lambda i: (0, 0)),
            pl.BlockSpec((1, h), lambda i: (0, 0)),
            pl.BlockSpec((h, d_out), lambda i: (0, 0)),
            pl.BlockSpec((1, d_out), lambda i: (0, 0)),
            pl.BlockSpec((d_out, lat), lambda i: (0, 0)),
            pl.BlockSpec((1, lat), lambda i: (0, 0)),
            pl.BlockSpec((d_out, lat), lambda i: (0, 0)),
            pl.BlockSpec((1, lat), lambda i: (0, 0)),
            pl.BlockSpec((mb, lat), lambda i: (i, 0)),
        ],
        out_specs=[
            pl.BlockSpec((mb, lat), lambda i: (i, 0)),
            pl.BlockSpec((mb, lat), lambda i: (i, 0)),
            pl.BlockSpec((mb, lat), lambda i: (i, 0)),
        ],
        out_shape=[shp((n, lat)), shp((n, lat)), shp((n, lat))],
    )(x, agg, w1, b1.reshape(1, -1), w2, b2.reshape(1, -1),
      wm, bm.reshape(1, -1), wv, bv.reshape(1, -1), eps)


def kernel(eeg_nodes, eeg_idx, W1_0, b1_0, W2_0, b2_0, W1_1, b1_1, W2_1, b2_1,
           W1_2, b1_2, W2_2, b2_2, W1_3, b1_3, W2_3, b2_3, Wm, bm, Wv, bv):
    n = eeg_nodes.shape[0]
    src = eeg_idx[0]
    dst = eeg_idx[1]

    def seg(x):
        return jax.ops.segment_sum(x[src], dst, num_segments=n)

    x = _gin_mlp(eeg_nodes, seg(eeg_nodes), W1_0, b1_0, W2_0, b2_0, act=True)
    x = _gin_mlp(x, seg(x), W1_1, b1_1, W2_1, b2_1, act=True)
    x = _gin_mlp(x, seg(x), W1_2, b1_2, W2_2, b2_2, act=True)
    eps = jax.random.normal(jax.random.key(1234), (n, Wm.shape[1]),
                            dtype=jnp.float32)
    z, mean, var = _final_layer(x, seg(x), W1_3, b1_3, W2_3, b2_3,
                                Wm, bm, Wv, bv, eps)
    return (z, mean, var)


# speedup vs baseline: 4.0322x; 4.0302x over previous
import functools

import jax
import jax.numpy as jnp
from jax import lax
from jax.experimental import pallas as pl
from jax.experimental.pallas import tpu as pltpu
from jax.experimental.pallas import tpu_sc as plsc

_LANES = 16
_TRASH = 64


def _prefix16(v):
    iota = lax.iota(jnp.int32, _LANES)
    for k in (1, 2, 4, 8):
        idx = jnp.maximum(iota - k, 0)
        shifted = lax.gather(
            v, idx[:, None],
            dimension_numbers=lax.GatherDimensionNumbers(
                offset_dims=(), collapsed_slice_dims=(0,),
                start_index_map=(0,)),
            slice_sizes=(1,),
            mode=lax.GatherScatterMode.PROMISE_IN_BOUNDS)
        v = v + jnp.where(iota >= k, shifted, 0)
    return v


def _chunk_size(ept):
    for ch in (2048, 2000, 1600, 1280, 1040, 1000, 800, 640, 400, 320, 208,
               160, 128, 80, 64, 48, 32, 16):
        if ept % ch == 0 and ch % _LANES == 0:
            return ch
    return _LANES


_ROWS_PER_TILE = 8


def _sc_build_adjacency(src, dst, np_rows):
    e_total = src.shape[0]
    info = plsc.get_sparse_core_info()
    nc, ns = info.num_cores, info.num_subcores
    ept = e_total // ns
    ch = _chunk_size(ept)
    nch = ept // ch
    half = np_rows // nc
    prows = ns * _ROWS_PER_TILE
    n_pass = half // prows
    lch = 2048
    cap = -(-ept // lch) * lch
    mesh = plsc.VectorSubcoreMesh(core_axis_name="c", subcore_axis_name="s")

    @functools.partial(
        pl.kernel,
        mesh=mesh,
        compiler_params=pltpu.CompilerParams(needs_layout_passes=False),
        out_type=jax.ShapeDtypeStruct((np_rows, np_rows), jnp.float32),
        scratch_types=[
            pltpu.VMEM((ch,), jnp.int32),
            pltpu.VMEM((ch,), jnp.int32),
            pltpu.VMEM((cap + _LANES,), jnp.int32),
            pltpu.VMEM((_LANES,), jnp.int32),
            pltpu.VMEM((ns * _LANES,), jnp.int32),
            pltpu.VMEM((lch,), jnp.int32),
            pltpu.VMEM((_ROWS_PER_TILE, np_rows), jnp.float32),
            pltpu.VMEM_SHARED((ns * cap,), jnp.int32),
            pltpu.VMEM_SHARED((ns * _LANES,), jnp.int32),
        ],
    )
    def adj_kernel(src_h, dst_h, zeros_h, a_h,
                   srcbuf, dstbuf, pend, cbuf, cntbuf, lbuf, strip,
                   lists_sh, counts_sh):
        c = lax.axis_index("c")
        s = lax.axis_index("s")
        ebase = s * ept
        lane_iota = lax.iota(jnp.int32, _LANES)
        one16 = jnp.ones((_LANES,), jnp.float32)

        def do_pass(p, _):
            base = c * half + p * prows
            pltpu.sync_copy(zeros_h, strip)

            def do_chunk(ci, cnt):
                off = ebase + ci * ch
                pltpu.sync_copy(src_h.at[pl.ds(off, ch)], srcbuf)
                pltpu.sync_copy(dst_h.at[pl.ds(off, ch)], dstbuf)

                def grp(g, cnt):
                    sv = srcbuf[pl.ds(g * _LANES, _LANES)]
                    dv = dstbuf[pl.ds(g * _LANES, _LANES)]
                    row = dv - base
                    inb = (row >= 0) & (row < prows)
                    ones = jnp.where(inb, 1, 0).astype(jnp.int32)
                    cs = _prefix16(ones)
                    pos = cnt + cs - 1
                    plsc.store_scatter(pend, [pos], (row << 14) | sv,
                                       mask=inb)
                    return cnt + cs[_LANES - 1]

                return lax.fori_loop(0, ch // _LANES, grp, cnt)

            cnt = lax.fori_loop(0, nch, do_chunk, jnp.int32(0))
            pltpu.sync_copy(pend.at[pl.ds(0, cap)],
                            lists_sh.at[pl.ds(s * cap, cap)])
            cbuf[pl.ds(0, _LANES)] = jnp.broadcast_to(cnt, (_LANES,))
            pltpu.sync_copy(cbuf, counts_sh.at[pl.ds(s * _LANES, _LANES)])
            plsc.subcore_barrier()

            pltpu.sync_copy(counts_sh, cntbuf)
            for k in range(ns):
                cnt_k = cntbuf[pl.ds(k * _LANES, _LANES)][0]

                def do_lchunk(li, _):
                    pltpu.sync_copy(
                        lists_sh.at[pl.ds(k * cap + li * lch, lch)], lbuf)

                    def lgrp(g, _):
                        gpos = li * lch + g * _LANES + lane_iota
                        val = lbuf[pl.ds(g * _LANES, _LANES)]
                        row = val >> 14
                        mine = ((row >> 3) == s) & (gpos < cnt_k)
                        plsc.addupdate_scatter(
                            strip, [row & 7, val & 16383], one16, mask=mine)
                        return _

                    lax.fori_loop(0, lch // _LANES, lgrp, None)
                    return _

                nlc = (cnt_k + lch - 1) // lch
                lax.fori_loop(0, nlc, do_lchunk, None)
            plsc.subcore_barrier()
            pltpu.sync_copy(strip,
                            a_h.at[pl.ds(base + s * _ROWS_PER_TILE,
                                         _ROWS_PER_TILE)])
            return _

        lax.fori_loop(0, n_pass, do_pass, None)

    zeros = jnp.zeros((_ROWS_PER_TILE, np_rows), jnp.float32)
    return adj_kernel(src, dst, zeros)


def _cast_kernel(a_ref, o_ref):
    o_ref[...] = a_ref[...].astype(jnp.bfloat16)


def _cast_bf16(a, mb=128):
    n, m = a.shape
    return pl.pallas_call(
        _cast_kernel,
        grid=(n // mb,),
        in_specs=[pl.BlockSpec((mb, m), lambda i: (i, 0))],
        out_specs=pl.BlockSpec((mb, m), lambda i: (i, 0)),
        out_shape=jax.ShapeDtypeStruct((n, m), jnp.bfloat16),
    )(a)


def _matmul_kernel(a_ref, x_ref, o_ref):
    @pl.when(pl.program_id(1) == 0)
    def _():
        o_ref[...] = jnp.zeros_like(o_ref)
    xb = x_ref[...].astype(jnp.bfloat16)
    o_ref[...] += jnp.dot(a_ref[...], xb, preferred_element_type=jnp.float32)


def _seg_matmul(a16, x, mb=256, kb=2048):
    n = a16.shape[0]
    d = x.shape[1]
    return pl.pallas_call(
        _matmul_kernel,
        grid=(n // mb, n // kb),
        in_specs=[
            pl.BlockSpec((mb, kb), lambda i, k: (i, k)),
            pl.BlockSpec((kb, d), lambda i, k: (k, 0)),
        ],
        out_specs=pl.BlockSpec((mb, d), lambda i, k: (i, 0)),
        out_shape=jax.ShapeDtypeStruct((n, d), jnp.float32),
        compiler_params=pltpu.CompilerParams(
            dimension_semantics=("arbitrary", "arbitrary")),
    )(a16, x)


def _gin_mlp_kernel(x_ref, a_ref, w1_ref, b1_ref, w2_ref, b2_ref, o_ref, *, act):
    h = x_ref[...] + a_ref[...]
    h = jnp.dot(h, w1_ref[...], preferred_element_type=jnp.float32) + b1_ref[...]
    h = jnp.maximum(h, 0.0)
    y = jnp.dot(h, w2_ref[...], preferred_element_type=jnp.float32) + b2_ref[...]
    if act:
        y = jnp.maximum(y, 0.0)
    o_ref[...] = y


def _gin_mlp(x, agg, w1, b1, w2, b2, act, mb=256):
    n, d_in = x.shape
    h = w1.shape[1]
    d_out = w2.shape[1]
    return pl.pallas_call(
        functools.partial(_gin_mlp_kernel, act=act),
        grid=(n // mb,),
        in_specs=[
            pl.BlockSpec((mb, d_in), lambda i: (i, 0)),
            pl.BlockSpec((mb, d_in), lambda i: (i, 0)),
            pl.BlockSpec((d_in, h), lambda i: (0, 0)),
            pl.BlockSpec((1, h), lambda i: (0, 0)),
            pl.BlockSpec((h, d_out), lambda i: (0, 0)),
            pl.BlockSpec((1, d_out), lambda i: (0, 0)),
        ],
        out_specs=pl.BlockSpec((mb, d_out), lambda i: (i, 0)),
        out_shape=jax.ShapeDtypeStruct((n, d_out), jnp.float32),
    )(x, agg, w1, b1.reshape(1, -1), w2, b2.reshape(1, -1))


def _final_kernel(x_ref, a_ref, w1_ref, b1_ref, w2_ref, b2_ref,
                  wm_ref, bm_ref, wv_ref, bv_ref, eps_ref,
                  z_ref, m_ref, v_ref):
    h = x_ref[...] + a_ref[...]
    h = jnp.dot(h, w1_ref[...], preferred_element_type=jnp.float32) + b1_ref[...]
    h = jnp.maximum(h, 0.0)
    y = jnp.dot(h, w2_ref[...], preferred_element_type=jnp.float32) + b2_ref[...]
    mean = jnp.dot(y, wm_ref[...], preferred_element_type=jnp.float32) + bm_ref[...]
    var = jnp.dot(y, wv_ref[...], preferred_element_type=jnp.float32) + bv_ref[...]
    m_ref[...] = mean
    v_ref[...] = var
    z_ref[...] = mean + var * eps_ref[...]


def _final_layer(x, agg, w1, b1, w2, b2, wm, bm, wv, bv, eps, mb=256):
    n, d_in = x.shape
    h = w1.shape[1]
    d_out = w2.shape[1]
    lat = wm.shape[1]
    shp = functools.partial(jax.ShapeDtypeStruct, dtype=jnp.float32)
    return pl.pallas_call(
        _final_kernel,
        grid=(n // mb,),
        in_specs=[
            pl.BlockSpec((mb, d_in), lambda i: (i, 0)),
            pl.BlockSpec((mb, d_in), lambda i: (i, 0)),
            pl.BlockSpec((d_in, h), lambda i: (0, 0)),
            pl.BlockSpec((1, h), lambda i: (0, 0)),
            pl.BlockSpec((h, d_out), lambda i: (0, 0)),
            pl.BlockSpec((1, d_out), lambda i: (0, 0)),
            pl.BlockSpec((d_out, lat), lambda i: (0, 0)),
            pl.BlockSpec((1, lat), lambda i: (0, 0)),
            pl.BlockSpec((d_out, lat), lambda i: (0, 0)),
            pl.BlockSpec((1, lat), lambda i: (0, 0)),
            pl.BlockSpec((mb, lat), lambda i: (i, 0)),
        ],
        out_specs=[
            pl.BlockSpec((mb, lat), lambda i: (i, 0)),
            pl.BlockSpec((mb, lat), lambda i: (i, 0)),
            pl.BlockSpec((mb, lat), lambda i: (i, 0)),
        ],
        out_shape=[shp((n, lat)), shp((n, lat)), shp((n, lat))],
    )(x, agg, w1, b1.reshape(1, -1), w2, b2.reshape(1, -1),
      wm, bm.reshape(1, -1), wv, bv.reshape(1, -1), eps)


def kernel(eeg_nodes, eeg_idx, W1_0, b1_0, W2_0, b2_0, W1_1, b1_1, W2_1, b2_1,
           W1_2, b1_2, W2_2, b2_2, W1_3, b1_3, W2_3, b2_3, Wm, bm, Wv, bv):
    n = eeg_nodes.shape[0]
    npad = -(-n // 512) * 512
    src = eeg_idx[0]
    dst = eeg_idx[1]
    x = jnp.pad(eeg_nodes, ((0, npad - n), (0, 0)))

    a16 = _cast_bf16(_sc_build_adjacency(src, dst, npad))

    def seg(xp):
        return _seg_matmul(a16, xp)

    x = _gin_mlp(x, seg(x), W1_0, b1_0, W2_0, b2_0, act=True)
    x = _gin_mlp(x, seg(x), W1_1, b1_1, W2_1, b2_1, act=True)
    x = _gin_mlp(x, seg(x), W1_2, b1_2, W2_2, b2_2, act=True)
    eps = jax.random.normal(jax.random.key(1234), (n, Wm.shape[1]),
                            dtype=jnp.float32)
    eps = jnp.pad(eps, ((0, npad - n), (0, 0)))
    z, mean, var = _final_layer(x, seg(x), W1_3, b1_3, W2_3, b2_3,
                                Wm, bm, Wv, bv, eps)
    return (z[:n], mean[:n], var[:n])


# speedup vs baseline: 5.6213x; 1.3941x over previous
import functools

import jax
import jax.numpy as jnp
from jax import lax
from jax.experimental import pallas as pl
from jax.experimental.pallas import tpu as pltpu
from jax.experimental.pallas import tpu_sc as plsc

_LANES = 16
_TRASH = 64


def _prefix16(v):
    iota = lax.iota(jnp.int32, _LANES)
    for k in (1, 2, 4, 8):
        idx = jnp.maximum(iota - k, 0)
        shifted = lax.gather(
            v, idx[:, None],
            dimension_numbers=lax.GatherDimensionNumbers(
                offset_dims=(), collapsed_slice_dims=(0,),
                start_index_map=(0,)),
            slice_sizes=(1,),
            mode=lax.GatherScatterMode.PROMISE_IN_BOUNDS)
        v = v + jnp.where(iota >= k, shifted, 0)
    return v


def _chunk_size(ept):
    for ch in (2048, 2000, 1600, 1280, 1040, 1000, 800, 640, 400, 320, 208,
               160, 128, 80, 64, 48, 32, 16):
        if ept % ch == 0 and ch % _LANES == 0:
            return ch
    return _LANES


_ROWS_PER_TILE = 8


def _sc_build_adjacency(src, dst, np_rows):
    e_total = src.shape[0]
    info = plsc.get_sparse_core_info()
    nc, ns = info.num_cores, info.num_subcores
    ept = e_total // ns
    ch = _chunk_size(ept)
    nch = ept // ch
    half = np_rows // nc
    prows = ns * _ROWS_PER_TILE
    n_pass = half // prows
    lch = 2048
    cap = -(-ept // lch) * lch
    mesh = plsc.VectorSubcoreMesh(core_axis_name="c", subcore_axis_name="s")

    @functools.partial(
        pl.kernel,
        mesh=mesh,
        compiler_params=pltpu.CompilerParams(needs_layout_passes=False),
        out_type=jax.ShapeDtypeStruct((np_rows, np_rows), jnp.float32),
        scratch_types=[
            pltpu.VMEM((ch,), jnp.int32),
            pltpu.VMEM((ch,), jnp.int32),
            pltpu.VMEM((cap + _LANES,), jnp.int32),
            pltpu.VMEM((_LANES,), jnp.int32),
            pltpu.VMEM((ns * _LANES,), jnp.int32),
            pltpu.VMEM((lch,), jnp.int32),
            pltpu.VMEM((_ROWS_PER_TILE, np_rows), jnp.float32),
            pltpu.VMEM_SHARED((ns * cap,), jnp.int32),
            pltpu.VMEM_SHARED((ns * _LANES,), jnp.int32),
        ],
    )
    def adj_kernel(src_h, dst_h, zeros_h, a_h,
                   srcbuf, dstbuf, pend, cbuf, cntbuf, lbuf, strip,
                   lists_sh, counts_sh):
        c = lax.axis_index("c")
        s = lax.axis_index("s")
        ebase = s * ept
        lane_iota = lax.iota(jnp.int32, _LANES)
        one16 = jnp.ones((_LANES,), jnp.float32)

        def do_pass(p, _):
            base = c * half + p * prows
            pltpu.sync_copy(zeros_h, strip)

            def do_chunk(ci, cnt):
                off = ebase + ci * ch
                pltpu.sync_copy(src_h.at[pl.ds(off, ch)], srcbuf)
                pltpu.sync_copy(dst_h.at[pl.ds(off, ch)], dstbuf)

                def grp(g, cnt):
                    sv = srcbuf[pl.ds(g * _LANES, _LANES)]
                    dv = dstbuf[pl.ds(g * _LANES, _LANES)]
                    row = dv - base
                    inb = (row >= 0) & (row < prows)
                    ones = jnp.where(inb, 1, 0).astype(jnp.int32)
                    cs = _prefix16(ones)
                    pos = cnt + cs - 1
                    plsc.store_scatter(pend, [pos], (row << 14) | sv,
                                       mask=inb)
                    return cnt + cs[_LANES - 1]

                return lax.fori_loop(0, ch // _LANES, grp, cnt)

            cnt = lax.fori_loop(0, nch, do_chunk, jnp.int32(0))
            pltpu.sync_copy(pend.at[pl.ds(0, cap)],
                            lists_sh.at[pl.ds(s * cap, cap)])
            cbuf[pl.ds(0, _LANES)] = jnp.broadcast_to(cnt, (_LANES,))
            pltpu.sync_copy(cbuf, counts_sh.at[pl.ds(s * _LANES, _LANES)])
            plsc.subcore_barrier()

            pltpu.sync_copy(counts_sh, cntbuf)
            for k in range(ns):
                cnt_k = cntbuf[pl.ds(k * _LANES, _LANES)][0]

                def do_lchunk(li, _):
                    pltpu.sync_copy(
                        lists_sh.at[pl.ds(k * cap + li * lch, lch)], lbuf)

                    def lgrp(g, _):
                        gpos = li * lch + g * _LANES + lane_iota
                        val = lbuf[pl.ds(g * _LANES, _LANES)]
                        row = val >> 14
                        mine = ((row >> 3) == s) & (gpos < cnt_k)
                        plsc.addupdate_scatter(
                            strip, [row & 7, val & 16383], one16, mask=mine)
                        return _

                    lax.fori_loop(0, lch // _LANES, lgrp, None)
                    return _

                nlc = (cnt_k + lch - 1) // lch
                lax.fori_loop(0, nlc, do_lchunk, None)
            plsc.subcore_barrier()
            pltpu.sync_copy(strip,
                            a_h.at[pl.ds(base + s * _ROWS_PER_TILE,
                                         _ROWS_PER_TILE)])
            return _

        lax.fori_loop(0, n_pass, do_pass, None)

    zeros = jnp.zeros((_ROWS_PER_TILE, np_rows), jnp.float32)
    return adj_kernel(src, dst, zeros)


def _cast_kernel(a_ref, o_ref):
    o_ref[...] = a_ref[...].astype(jnp.bfloat16)


def _cast_bf16(a, mb=128):
    n, m = a.shape
    return pl.pallas_call(
        _cast_kernel,
        grid=(n // mb,),
        in_specs=[pl.BlockSpec((mb, m), lambda i: (i, 0))],
        out_specs=pl.BlockSpec((mb, m), lambda i: (i, 0)),
        out_shape=jax.ShapeDtypeStruct((n, m), jnp.bfloat16),
    )(a)


def _matmul_kernel(a_ref, x_ref, o_ref):
    @pl.when(pl.program_id(1) == 0)
    def _():
        o_ref[...] = jnp.zeros_like(o_ref)
    o_ref[...] += jnp.dot(a_ref[...], x_ref[...],
                          preferred_element_type=jnp.float32)


def _seg_matmul(a16, x16, mb=1024, kb=2048):
    n = a16.shape[0]
    d = x16.shape[1]
    return pl.pallas_call(
        _matmul_kernel,
        grid=(n // mb, n // kb),
        in_specs=[
            pl.BlockSpec((mb, kb), lambda i, k: (i, k)),
            pl.BlockSpec((kb, d), lambda i, k: (k, 0)),
        ],
        out_specs=pl.BlockSpec((mb, d), lambda i, k: (i, 0)),
        out_shape=jax.ShapeDtypeStruct((n, d), jnp.float32),
        compiler_params=pltpu.CompilerParams(
            dimension_semantics=("arbitrary", "arbitrary")),
    )(a16, x16)


def _gin_mlp_kernel(x_ref, a_ref, w1_ref, b1_ref, w2_ref, b2_ref,
                    o_ref, o16_ref):
    h = (x_ref[...] + a_ref[...]).astype(jnp.bfloat16)
    h = jnp.dot(h, w1_ref[...], preferred_element_type=jnp.float32) + b1_ref[...]
    h = jnp.maximum(h, 0.0).astype(jnp.bfloat16)
    y = jnp.dot(h, w2_ref[...], preferred_element_type=jnp.float32) + b2_ref[...]
    y = jnp.maximum(y, 0.0)
    o_ref[...] = y
    o16_ref[...] = y.astype(jnp.bfloat16)


def _gin_mlp(x, agg, w1, b1, w2, b2, mb=256):
    n, d_in = x.shape
    h = w1.shape[1]
    d_out = w2.shape[1]
    return pl.pallas_call(
        _gin_mlp_kernel,
        grid=(n // mb,),
        in_specs=[
            pl.BlockSpec((mb, d_in), lambda i: (i, 0)),
            pl.BlockSpec((mb, d_in), lambda i: (i, 0)),
            pl.BlockSpec((d_in, h), lambda i: (0, 0)),
            pl.BlockSpec((1, h), lambda i: (0, 0)),
            pl.BlockSpec((h, d_out), lambda i: (0, 0)),
            pl.BlockSpec((1, d_out), lambda i: (0, 0)),
        ],
        out_specs=[
            pl.BlockSpec((mb, d_out), lambda i: (i, 0)),
            pl.BlockSpec((mb, d_out), lambda i: (i, 0)),
        ],
        out_shape=[jax.ShapeDtypeStruct((n, d_out), jnp.float32),
                   jax.ShapeDtypeStruct((n, d_out), jnp.bfloat16)],
    )(x, agg, w1, b1.reshape(1, -1), w2, b2.reshape(1, -1))


def _final_kernel(x_ref, a_ref, w1_ref, b1_ref, w2_ref, b2_ref,
                  wm_ref, bm_ref, wv_ref, bv_ref, eps_ref,
                  z_ref, m_ref, v_ref):
    h = (x_ref[...] + a_ref[...]).astype(jnp.bfloat16)
    h = jnp.dot(h, w1_ref[...], preferred_element_type=jnp.float32) + b1_ref[...]
    h = jnp.maximum(h, 0.0).astype(jnp.bfloat16)
    y = jnp.dot(h, w2_ref[...], preferred_element_type=jnp.float32) + b2_ref[...]
    mean = jnp.dot(y, wm_ref[...], preferred_element_type=jnp.float32) + bm_ref[...]
    var = jnp.dot(y, wv_ref[...], preferred_element_type=jnp.float32) + bv_ref[...]
    m_ref[...] = mean
    v_ref[...] = var
    z_ref[...] = mean + var * eps_ref[...]


def _final_layer(x, agg, w1, b1, w2, b2, wm, bm, wv, bv, eps, mb=256):
    n, d_in = x.shape
    h = w1.shape[1]
    d_out = w2.shape[1]
    lat = wm.shape[1]
    shp = functools.partial(jax.ShapeDtypeStruct, dtype=jnp.float32)
    return pl.pallas_call(
        _final_kernel,
        grid=(n // mb,),
        in_specs=[
            pl.BlockSpec((mb, d_in), lambda i: (i, 0)),
            pl.BlockSpec((mb, d_in), lambda i: (i, 0)),
            pl.BlockSpec((d_in, h), lambda i: (0, 0)),
            pl.BlockSpec((1, h), lambda i: (0, 0)),
            pl.BlockSpec((h, d_out), lambda i: (0, 0)),
            pl.BlockSpec((1, d_out), lambda i: (0, 0)),
            pl.BlockSpec((d_out, lat), lambda i: (0, 0)),
            pl.BlockSpec((1, lat), lambda i: (0, 0)),
            pl.BlockSpec((d_out, lat), lambda i: (0, 0)),
            pl.BlockSpec((1, lat), lambda i: (0, 0)),
            pl.BlockSpec((mb, lat), lambda i: (i, 0)),
        ],
        out_specs=[
            pl.BlockSpec((mb, lat), lambda i: (i, 0)),
            pl.BlockSpec((mb, lat), lambda i: (i, 0)),
            pl.BlockSpec((mb, lat), lambda i: (i, 0)),
        ],
        out_shape=[shp((n, lat)), shp((n, lat)), shp((n, lat))],
    )(x, agg, w1, b1.reshape(1, -1), w2, b2.reshape(1, -1),
      wm, bm.reshape(1, -1), wv, bv.reshape(1, -1), eps)


def kernel(eeg_nodes, eeg_idx, W1_0, b1_0, W2_0, b2_0, W1_1, b1_1, W2_1, b2_1,
           W1_2, b1_2, W2_2, b2_2, W1_3, b1_3, W2_3, b2_3, Wm, bm, Wv, bv):
    n = eeg_nodes.shape[0]
    npad = -(-n // 512) * 512
    src = eeg_idx[0]
    dst = eeg_idx[1]
    x = jnp.pad(eeg_nodes, ((0, npad - n), (0, 0)))

    a16 = _cast_bf16(_sc_build_adjacency(src, dst, npad))
    x16 = _cast_bf16(x, mb=512)

    x, x16 = _gin_mlp(x, _seg_matmul(a16, x16),
                      W1_0.astype(jnp.bfloat16), b1_0,
                      W2_0.astype(jnp.bfloat16), b2_0)
    x, x16 = _gin_mlp(x, _seg_matmul(a16, x16),
                      W1_1.astype(jnp.bfloat16), b1_1,
                      W2_1.astype(jnp.bfloat16), b2_1)
    x, x16 = _gin_mlp(x, _seg_matmul(a16, x16),
                      W1_2.astype(jnp.bfloat16), b1_2,
                      W2_2.astype(jnp.bfloat16), b2_2)
    eps = jax.random.normal(jax.random.key(1234), (n, Wm.shape[1]),
                            dtype=jnp.float32)
    eps = jnp.pad(eps, ((0, npad - n), (0, 0)))
    z, mean, var = _final_layer(x, _seg_matmul(a16, x16),
                                W1_3.astype(jnp.bfloat16), b1_3,
                                W2_3.astype(jnp.bfloat16), b2_3,
                                Wm, bm, Wv, bv, eps)
    return (z[:n], mean[:n], var[:n])
